# baseline (speedup 1.0000x reference)
import jax
import jax.numpy as jnp
from jax import lax
from jax.experimental import pallas as pl
from jax.experimental.pallas import tpu as pltpu
from jax.experimental.pallas import tpu_sc as plsc

N = 100000
E = 6400000
EPS = 1e-06
LAMB = 1.0
X0 = 0.05

NSUB = 16
NCORE = 2
NTILE = NSUB * NCORE
C = 2000
CHUNKS = E // C
NCH = CHUNKS // NTILE
STRIPE = 6272
NPAD = NSUB * STRIPE
F32 = jnp.float32
I32 = jnp.int32


def _step_body(acc_in, edge_hbm, ea_hbm, acc_out,
               sig_sh, acc_sh,
               n0_v, n1_v, d0_v, d1_v, zero_v,
               src0, src1, src2, idx0, idx1, idx2,
               eav0, eav1, eav2, xj0, xj1, xj2, xi0, xi1, xi2,
               vw0, vw1, vw2,
               sl0, sl1, sl2, sg0, sg1, sg2, ss0, ss1, ss2):
    src_v = (src0, src1, src2)
    idx_v = (idx0, idx1, idx2)
    ea_v = (eav0, eav1, eav2)
    xj_v = (xj0, xj1, xj2)
    xi_v = (xi0, xi1, xi2)
    vw_v = (vw0, vw1, vw2)
    sem_l = (sl0, sl1, sl2)
    sem_g = (sg0, sg1, sg2)
    sem_s = (ss0, ss1, ss2)
    c = lax.axis_index("c")
    s = lax.axis_index("s")
    g = c * NSUB + s
    st = pl.ds(s * STRIPE, STRIPE)

    pltpu.sync_copy(acc_in.at[pl.ds(s * STRIPE, STRIPE)], d0_v)
    pltpu.sync_copy(acc_in.at[pl.ds(NPAD + s * STRIPE, STRIPE)], n0_v)
    pltpu.sync_copy(acc_in.at[pl.ds(2 * NPAD + s * STRIPE, STRIPE)], d1_v)
    pltpu.sync_copy(acc_in.at[pl.ds(3 * NPAD + s * STRIPE, STRIPE)], n1_v)

    def comb(i, carry):
        sl = pl.ds(i * 16, 16)
        numv = n0_v[sl] + n1_v[sl] + X0
        denv = d0_v[sl] + d1_v[sl] + LAMB
        n0_v[sl] = numv / denv
        zero_v[sl] = jnp.zeros((16,), F32)
        return carry

    lax.fori_loop(0, STRIPE // 16, comb, 0)

    pltpu.sync_copy(n0_v, sig_sh.at[st])
    pltpu.sync_copy(zero_v, acc_sh.at[st])
    pltpu.sync_copy(zero_v, acc_sh.at[pl.ds(NPAD + s * STRIPE, STRIPE)])
    plsc.subcore_barrier()

    def issue_load(t, b):
        base = (g + t * NTILE) * C
        pltpu.async_copy(edge_hbm.at[pl.ds(base, C)], src_v[b], sem_l[b])
        pltpu.async_copy(edge_hbm.at[pl.ds(E + base, C)],
                         idx_v[b].at[pl.ds(0, C)], sem_l[b])
        pltpu.async_copy(ea_hbm.at[pl.ds(base, C)], ea_v[b], sem_l[b])

    def wait_load(t, b):
        base = (g + t * NTILE) * C
        pltpu.make_async_copy(edge_hbm.at[pl.ds(base, C)], src_v[b],
                              sem_l[b]).wait()
        pltpu.make_async_copy(edge_hbm.at[pl.ds(E + base, C)],
                              idx_v[b].at[pl.ds(0, C)], sem_l[b]).wait()
        pltpu.make_async_copy(ea_hbm.at[pl.ds(base, C)], ea_v[b],
                              sem_l[b]).wait()

    def issue_gather(b):
        pltpu.async_copy(sig_sh.at[src_v[b]], xj_v[b], sem_g[b])
        pltpu.async_copy(sig_sh.at[idx_v[b].at[pl.ds(0, C)]], xi_v[b],
                         sem_g[b])

    def wait_gather(b):
        pltpu.make_async_copy(sig_sh.at[src_v[b]], xj_v[b], sem_g[b]).wait()
        pltpu.make_async_copy(sig_sh.at[idx_v[b].at[pl.ds(0, C)]], xi_v[b],
                              sem_g[b]).wait()

    def issue_scatter(b):
        pltpu.async_copy(vw_v[b], acc_sh.at[idx_v[b]], sem_s[b], add=True)

    def wait_scatter(b):
        pltpu.make_async_copy(vw_v[b], acc_sh.at[idx_v[b]], sem_s[b]).wait()

    def compute(b):
        xjr, xir, ear, idxr, vwr = xj_v[b], xi_v[b], ea_v[b], idx_v[b], vw_v[b]
        npadv = jnp.full((16,), NPAD, I32)

        def vstep(j, carry2):
            o = j * 16
            sl = pl.ds(o, 16)
            sl2 = pl.ds(C + o, 16)
            xj = xjr[sl]
            xi = xir[sl]
            ea = ear[sl]
            d = xj - xi + EPS
            w = (ea * ea) * (d * d)
            vwr[sl] = w
            vwr[sl2] = w * xj
            idxr[sl2] = idxr[sl] + npadv
            return carry2

        lax.fori_loop(0, C // 16, vstep, 0)

    issue_load(0, 0)
    issue_load(1, 1)
    wait_load(0, 0)
    issue_gather(0)

    def pipe(i, carry):
        for b in range(3):
            t = i * 3 + b
            b1 = (b + 1) % 3
            b2 = (b + 2) % 3

            @pl.when(jnp.logical_and(t >= 1, t <= NCH))
            def _():
                wait_scatter(b2)

            @pl.when(t <= NCH - 3)
            def _():
                issue_load(t + 2, b2)

            @pl.when(t <= NCH - 2)
            def _():
                wait_load(t + 1, b1)
                issue_gather(b1)

            @pl.when(t <= NCH - 1)
            def _():
                wait_gather(b)
                compute(b)
                issue_scatter(b)
        return carry

    lax.fori_loop(0, (NCH + 3) // 3, pipe, 0)

    plsc.subcore_barrier()
    pltpu.sync_copy(acc_sh.at[st],
                    acc_out.at[pl.ds(2 * NPAD * c + s * STRIPE, STRIPE)])
    pltpu.sync_copy(acc_sh.at[pl.ds(NPAD + s * STRIPE, STRIPE)],
                    acc_out.at[pl.ds(2 * NPAD * c + NPAD + s * STRIPE,
                                     STRIPE)])


def _fin_body(acc_in, out_hbm, n0_v, n1_v, d0_v, d1_v):
    c = lax.axis_index("c")
    s = lax.axis_index("s")

    @pl.when(c == 0)
    def _():
        st = pl.ds(s * STRIPE, STRIPE)
        pltpu.sync_copy(acc_in.at[pl.ds(s * STRIPE, STRIPE)], d0_v)
        pltpu.sync_copy(acc_in.at[pl.ds(NPAD + s * STRIPE, STRIPE)], n0_v)
        pltpu.sync_copy(acc_in.at[pl.ds(2 * NPAD + s * STRIPE, STRIPE)], d1_v)
        pltpu.sync_copy(acc_in.at[pl.ds(3 * NPAD + s * STRIPE, STRIPE)], n1_v)

        def comb(i, carry):
            sl = pl.ds(i * 16, 16)
            numv = n0_v[sl] + n1_v[sl] + X0
            denv = d0_v[sl] + d1_v[sl] + LAMB
            n0_v[sl] = numv / denv
            return carry

        lax.fori_loop(0, STRIPE // 16, comb, 0)
        pltpu.sync_copy(n0_v, out_hbm.at[st])


def kernel(signal, edge_attr, edge_index, itr):
    sig = jnp.pad(signal.reshape(N), (0, NPAD - N))
    edge1 = edge_index.reshape(2 * E)
    ea1 = edge_attr.reshape(E)

    z = jnp.zeros((NPAD,), F32)
    acc0 = jnp.concatenate([z, sig - X0, z, z])

    mesh = plsc.VectorSubcoreMesh(core_axis_name="c", subcore_axis_name="s")
    step = pl.kernel(
        _step_body,
        out_type=jax.ShapeDtypeStruct((4 * NPAD,), F32),
        mesh=mesh,
        scratch_types=[
            pltpu.VMEM_SHARED((NPAD,), F32),
            pltpu.VMEM_SHARED((2 * NPAD,), F32),
            pltpu.VMEM((STRIPE,), F32),
            pltpu.VMEM((STRIPE,), F32),
            pltpu.VMEM((STRIPE,), F32),
            pltpu.VMEM((STRIPE,), F32),
            pltpu.VMEM((STRIPE,), F32),
        ] + [pltpu.VMEM((C,), I32) for _ in range(3)]
          + [pltpu.VMEM((2 * C,), I32) for _ in range(3)]
          + [pltpu.VMEM((C,), F32) for _ in range(9)]
          + [pltpu.VMEM((2 * C,), F32) for _ in range(3)]
          + [pltpu.SemaphoreType.DMA for _ in range(9)],
    )

    def body(_, acc):
        return step(acc, edge1, ea1)

    acc_f = lax.fori_loop(0, itr, body, acc0)

    fin = pl.kernel(
        _fin_body,
        out_type=jax.ShapeDtypeStruct((NPAD,), F32),
        mesh=mesh,
        scratch_types=[
            pltpu.VMEM((STRIPE,), F32),
            pltpu.VMEM((STRIPE,), F32),
            pltpu.VMEM((STRIPE,), F32),
            pltpu.VMEM((STRIPE,), F32),
        ],
    )
    sig_out = fin(acc_f)
    return sig_out[:N].reshape(N, 1)


# speedup vs baseline: 1.0523x; 1.0523x over previous
import jax
import jax.numpy as jnp
from jax import lax
from jax.experimental import pallas as pl
from jax.experimental.pallas import tpu as pltpu
from jax.experimental.pallas import tpu_sc as plsc

N = 100000
E = 6400000
EPS = 1e-06
LAMB = 1.0
X0 = 0.05

NSUB = 16
NCORE = 2
NTILE = NSUB * NCORE
C = 2000
CHUNKS = E // C
NCH = CHUNKS // NTILE
STRIPE = 6272
NPAD = NSUB * STRIPE
F32 = jnp.float32
I32 = jnp.int32


def _step_body(acc_in, edge_hbm, ea_hbm, acc_out,
               sig_sh, den_sh, num_sh,
               n0_v, n1_v, d0_v, d1_v, zero_v,
               src0, src1, src2, dst0, dst1, dst2,
               eav0, eav1, eav2, xj0, xj1, xj2, xi0, xi1, xi2,
               w0, w1, w2, wx0, wx1, wx2,
               sl0, sl1, sl2, sg0, sg1, sg2, ss0, ss1, ss2):
    src_v = (src0, src1, src2)
    dst_v = (dst0, dst1, dst2)
    ea_v = (eav0, eav1, eav2)
    xj_v = (xj0, xj1, xj2)
    xi_v = (xi0, xi1, xi2)
    w_v = (w0, w1, w2)
    wx_v = (wx0, wx1, wx2)
    sem_l = (sl0, sl1, sl2)
    sem_g = (sg0, sg1, sg2)
    sem_s = (ss0, ss1, ss2)
    c = lax.axis_index("c")
    s = lax.axis_index("s")
    g = c * NSUB + s
    st = pl.ds(s * STRIPE, STRIPE)

    pltpu.sync_copy(acc_in.at[pl.ds(s * STRIPE, STRIPE)], d0_v)
    pltpu.sync_copy(acc_in.at[pl.ds(NPAD + s * STRIPE, STRIPE)], n0_v)
    pltpu.sync_copy(acc_in.at[pl.ds(2 * NPAD + s * STRIPE, STRIPE)], d1_v)
    pltpu.sync_copy(acc_in.at[pl.ds(3 * NPAD + s * STRIPE, STRIPE)], n1_v)

    def comb(i, carry):
        sl = pl.ds(i * 16, 16)
        numv = n0_v[sl] + n1_v[sl] + X0
        denv = d0_v[sl] + d1_v[sl] + LAMB
        n0_v[sl] = numv / denv
        zero_v[sl] = jnp.zeros((16,), F32)
        return carry

    lax.fori_loop(0, STRIPE // 16, comb, 0)

    pltpu.sync_copy(n0_v, sig_sh.at[st])
    pltpu.sync_copy(zero_v, den_sh.at[st])
    pltpu.sync_copy(zero_v, num_sh.at[st])
    plsc.subcore_barrier()

    def issue_load(t, b):
        base = (g + t * NTILE) * C
        pltpu.async_copy(edge_hbm.at[pl.ds(base, C)], src_v[b], sem_l[b])
        pltpu.async_copy(edge_hbm.at[pl.ds(E + base, C)], dst_v[b], sem_l[b])
        pltpu.async_copy(ea_hbm.at[pl.ds(base, C)], ea_v[b], sem_l[b])

    def wait_load(t, b):
        base = (g + t * NTILE) * C
        pltpu.make_async_copy(edge_hbm.at[pl.ds(base, C)], src_v[b],
                              sem_l[b]).wait()
        pltpu.make_async_copy(edge_hbm.at[pl.ds(E + base, C)], dst_v[b],
                              sem_l[b]).wait()
        pltpu.make_async_copy(ea_hbm.at[pl.ds(base, C)], ea_v[b],
                              sem_l[b]).wait()

    def issue_gather(b):
        pltpu.async_copy(sig_sh.at[src_v[b]], xj_v[b], sem_g[b])
        pltpu.async_copy(sig_sh.at[dst_v[b]], xi_v[b], sem_g[b])

    def wait_gather(b):
        pltpu.make_async_copy(sig_sh.at[src_v[b]], xj_v[b], sem_g[b]).wait()
        pltpu.make_async_copy(sig_sh.at[dst_v[b]], xi_v[b], sem_g[b]).wait()

    def wait_num_scatter(b):
        pltpu.make_async_copy(wx_v[b], num_sh.at[dst_v[b]], sem_s[b]).wait()

    def compute(b):
        xjr, xir, ear, wr, wxr = xj_v[b], xi_v[b], ea_v[b], w_v[b], wx_v[b]

        def vstep(j, carry2):
            for k in range(4):
                sl = pl.ds(j * 64 + k * 16, 16)
                xj = xjr[sl]
                xi = xir[sl]
                ea = ear[sl]
                d = xj - xi + EPS
                w = (ea * ea) * (d * d)
                wr[sl] = w
                wxr[sl] = w * xj
            return carry2

        lax.fori_loop(0, C // 64, vstep, 0)

    issue_load(0, 0)
    issue_load(1, 1)
    wait_load(0, 0)
    issue_gather(0)

    def pipe(i, carry):
        for b in range(3):
            t = i * 3 + b
            b1 = (b + 1) % 3
            b2 = (b + 2) % 3

            @pl.when(jnp.logical_and(t >= 1, t <= NCH))
            def _():
                wait_num_scatter(b2)

            @pl.when(t <= NCH - 3)
            def _():
                issue_load(t + 2, b2)

            @pl.when(t <= NCH - 2)
            def _():
                wait_load(t + 1, b1)
                issue_gather(b1)

            @pl.when(t <= NCH - 1)
            def _():
                wait_gather(b)
                compute(b)
                cp_d = pltpu.async_copy(w_v[b], den_sh.at[dst_v[b]],
                                        sem_s[b], add=True)
                cp_d.wait()
                pltpu.async_copy(wx_v[b], num_sh.at[dst_v[b]],
                                 sem_s[b], add=True)
        return carry

    lax.fori_loop(0, (NCH + 3) // 3, pipe, 0)

    plsc.subcore_barrier()
    pltpu.sync_copy(den_sh.at[st],
                    acc_out.at[pl.ds(2 * NPAD * c + s * STRIPE, STRIPE)])
    pltpu.sync_copy(num_sh.at[st],
                    acc_out.at[pl.ds(2 * NPAD * c + NPAD + s * STRIPE,
                                     STRIPE)])


def _fin_body(acc_in, out_hbm, n0_v, n1_v, d0_v, d1_v):
    c = lax.axis_index("c")
    s = lax.axis_index("s")

    @pl.when(c == 0)
    def _():
        st = pl.ds(s * STRIPE, STRIPE)
        pltpu.sync_copy(acc_in.at[pl.ds(s * STRIPE, STRIPE)], d0_v)
        pltpu.sync_copy(acc_in.at[pl.ds(NPAD + s * STRIPE, STRIPE)], n0_v)
        pltpu.sync_copy(acc_in.at[pl.ds(2 * NPAD + s * STRIPE, STRIPE)], d1_v)
        pltpu.sync_copy(acc_in.at[pl.ds(3 * NPAD + s * STRIPE, STRIPE)], n1_v)

        def comb(i, carry):
            sl = pl.ds(i * 16, 16)
            numv = n0_v[sl] + n1_v[sl] + X0
            denv = d0_v[sl] + d1_v[sl] + LAMB
            n0_v[sl] = numv / denv
            return carry

        lax.fori_loop(0, STRIPE // 16, comb, 0)
        pltpu.sync_copy(n0_v, out_hbm.at[st])


def kernel(signal, edge_attr, edge_index, itr):
    sig = jnp.pad(signal.reshape(N), (0, NPAD - N))
    edge1 = edge_index.reshape(2 * E)
    ea1 = edge_attr.reshape(E)

    z = jnp.zeros((NPAD,), F32)
    acc0 = jnp.concatenate([z, sig - X0, z, z])

    mesh = plsc.VectorSubcoreMesh(core_axis_name="c", subcore_axis_name="s")
    step = pl.kernel(
        _step_body,
        out_type=jax.ShapeDtypeStruct((4 * NPAD,), F32),
        mesh=mesh,
        scratch_types=[
            pltpu.VMEM_SHARED((NPAD,), F32),
            pltpu.VMEM_SHARED((NPAD,), F32),
            pltpu.VMEM_SHARED((NPAD,), F32),
            pltpu.VMEM((STRIPE,), F32),
            pltpu.VMEM((STRIPE,), F32),
            pltpu.VMEM((STRIPE,), F32),
            pltpu.VMEM((STRIPE,), F32),
            pltpu.VMEM((STRIPE,), F32),
        ] + [pltpu.VMEM((C,), I32) for _ in range(6)]
          + [pltpu.VMEM((C,), F32) for _ in range(15)]
          + [pltpu.SemaphoreType.DMA for _ in range(9)],
    )

    def body(_, acc):
        return step(acc, edge1, ea1)

    acc_f = lax.fori_loop(0, itr, body, acc0)

    fin = pl.kernel(
        _fin_body,
        out_type=jax.ShapeDtypeStruct((NPAD,), F32),
        mesh=mesh,
        scratch_types=[
            pltpu.VMEM((STRIPE,), F32),
            pltpu.VMEM((STRIPE,), F32),
            pltpu.VMEM((STRIPE,), F32),
            pltpu.VMEM((STRIPE,), F32),
        ],
    )
    sig_out = fin(acc_f)
    return sig_out[:N].reshape(N, 1)


# speedup vs baseline: 1.0594x; 1.0068x over previous
import jax
import jax.numpy as jnp
from jax import lax
from jax.experimental import pallas as pl
from jax.experimental.pallas import tpu as pltpu
from jax.experimental.pallas import tpu_sc as plsc

N = 100000
E = 6400000
EPS = 1e-06
LAMB = 1.0
X0 = 0.05

NSUB = 16
NCORE = 2
NTILE = NSUB * NCORE
C = 2000
CHUNKS = E // C
NCH = CHUNKS // NTILE
STRIPE = 6272
NPAD = NSUB * STRIPE
F32 = jnp.float32
I32 = jnp.int32


def _step_body(acc_in, edge_hbm, ea_hbm, acc_out,
               sig_sh, den_sh, num_sh,
               n0_v, n1_v, d0_v, d1_v, zero_v,
               src0, src1, src2, dst0, dst1, dst2,
               eav0, eav1, eav2, xj0, xj1, xj2, xi0, xi1, xi2,
               w0, w1, w2, wx0, wx1, wx2,
               sl0, sl1, sl2, sg0, sg1, sg2, ss0, ss1, ss2):
    src_v = (src0, src1, src2)
    dst_v = (dst0, dst1, dst2)
    ea_v = (eav0, eav1, eav2)
    xj_v = (xj0, xj1, xj2)
    xi_v = (xi0, xi1, xi2)
    w_v = (w0, w1, w2)
    wx_v = (wx0, wx1, wx2)
    sem_l = (sl0, sl1, sl2)
    sem_g = (sg0, sg1, sg2)
    sem_s = (ss0, ss1, ss2)
    c = lax.axis_index("c")
    s = lax.axis_index("s")
    g = c * NSUB + s
    st = pl.ds(s * STRIPE, STRIPE)

    pltpu.sync_copy(acc_in.at[pl.ds(s * STRIPE, STRIPE)], d0_v)
    pltpu.sync_copy(acc_in.at[pl.ds(NPAD + s * STRIPE, STRIPE)], n0_v)
    pltpu.sync_copy(acc_in.at[pl.ds(2 * NPAD + s * STRIPE, STRIPE)], d1_v)
    pltpu.sync_copy(acc_in.at[pl.ds(3 * NPAD + s * STRIPE, STRIPE)], n1_v)

    def comb(i, carry):
        sl = pl.ds(i * 16, 16)
        numv = n0_v[sl] + n1_v[sl] + X0
        denv = d0_v[sl] + d1_v[sl] + LAMB
        n0_v[sl] = numv / denv
        zero_v[sl] = jnp.zeros((16,), F32)
        return carry

    lax.fori_loop(0, STRIPE // 16, comb, 0)

    pltpu.sync_copy(n0_v, sig_sh.at[st])
    pltpu.sync_copy(zero_v, den_sh.at[st])
    pltpu.sync_copy(zero_v, num_sh.at[st])
    plsc.subcore_barrier()

    def issue_load(t, b):
        base = (g + t * NTILE) * C
        pltpu.async_copy(edge_hbm.at[pl.ds(base, C)], src_v[b], sem_l[b])
        pltpu.async_copy(edge_hbm.at[pl.ds(E + base, C)], dst_v[b], sem_l[b])
        pltpu.async_copy(ea_hbm.at[pl.ds(base, C)], ea_v[b], sem_l[b])

    def wait_load(t, b):
        base = (g + t * NTILE) * C
        pltpu.make_async_copy(edge_hbm.at[pl.ds(base, C)], src_v[b],
                              sem_l[b]).wait()
        pltpu.make_async_copy(edge_hbm.at[pl.ds(E + base, C)], dst_v[b],
                              sem_l[b]).wait()
        pltpu.make_async_copy(ea_hbm.at[pl.ds(base, C)], ea_v[b],
                              sem_l[b]).wait()

    def issue_gather(b):
        pltpu.async_copy(sig_sh.at[src_v[b]], xj_v[b], sem_g[b])
        pltpu.async_copy(sig_sh.at[dst_v[b]], xi_v[b], sem_g[b])

    def wait_gather(b):
        pltpu.make_async_copy(sig_sh.at[src_v[b]], xj_v[b], sem_g[b]).wait()
        pltpu.make_async_copy(sig_sh.at[dst_v[b]], xi_v[b], sem_g[b]).wait()

    def compute(b):
        xjr, xir, ear, wr, wxr = xj_v[b], xi_v[b], ea_v[b], w_v[b], wx_v[b]

        def vstep(j, carry2):
            for k in range(4):
                sl = pl.ds(j * 64 + k * 16, 16)
                xj = xjr[sl]
                xi = xir[sl]
                ea = ear[sl]
                d = xj - xi + EPS
                w = (ea * ea) * (d * d)
                wr[sl] = w
                wxr[sl] = w * xj
            return carry2

        lax.fori_loop(0, C // 64, vstep, 0)

    issue_load(0, 0)
    issue_load(1, 1)
    wait_load(0, 0)
    issue_gather(0)

    def pipe(i, carry):
        for b in range(3):
            t = i * 3 + b
            b1 = (b + 1) % 3
            b2 = (b + 2) % 3

            @pl.when(t <= NCH - 3)
            def _():
                issue_load(t + 2, b2)

            @pl.when(t <= NCH - 2)
            def _():
                wait_load(t + 1, b1)
                issue_gather(b1)

            @pl.when(t <= NCH - 1)
            def _():
                wait_gather(b)
                compute(b)
                pltpu.sync_copy(w_v[b], den_sh.at[dst_v[b]], add=True)
                pltpu.sync_copy(wx_v[b], num_sh.at[dst_v[b]], add=True)
        return carry

    lax.fori_loop(0, (NCH + 3) // 3, pipe, 0)

    plsc.subcore_barrier()
    pltpu.sync_copy(den_sh.at[st],
                    acc_out.at[pl.ds(2 * NPAD * c + s * STRIPE, STRIPE)])
    pltpu.sync_copy(num_sh.at[st],
                    acc_out.at[pl.ds(2 * NPAD * c + NPAD + s * STRIPE,
                                     STRIPE)])


def _fin_body(acc_in, out_hbm, n0_v, n1_v, d0_v, d1_v):
    c = lax.axis_index("c")
    s = lax.axis_index("s")

    @pl.when(c == 0)
    def _():
        st = pl.ds(s * STRIPE, STRIPE)
        pltpu.sync_copy(acc_in.at[pl.ds(s * STRIPE, STRIPE)], d0_v)
        pltpu.sync_copy(acc_in.at[pl.ds(NPAD + s * STRIPE, STRIPE)], n0_v)
        pltpu.sync_copy(acc_in.at[pl.ds(2 * NPAD + s * STRIPE, STRIPE)], d1_v)
        pltpu.sync_copy(acc_in.at[pl.ds(3 * NPAD + s * STRIPE, STRIPE)], n1_v)

        def comb(i, carry):
            sl = pl.ds(i * 16, 16)
            numv = n0_v[sl] + n1_v[sl] + X0
            denv = d0_v[sl] + d1_v[sl] + LAMB
            n0_v[sl] = numv / denv
            return carry

        lax.fori_loop(0, STRIPE // 16, comb, 0)
        pltpu.sync_copy(n0_v, out_hbm.at[st])


def kernel(signal, edge_attr, edge_index, itr):
    sig = jnp.pad(signal.reshape(N), (0, NPAD - N))
    edge1 = edge_index.reshape(2 * E)
    ea1 = edge_attr.reshape(E)

    z = jnp.zeros((NPAD,), F32)
    acc0 = jnp.concatenate([z, sig - X0, z, z])

    mesh = plsc.VectorSubcoreMesh(core_axis_name="c", subcore_axis_name="s")
    step = pl.kernel(
        _step_body,
        out_type=jax.ShapeDtypeStruct((4 * NPAD,), F32),
        mesh=mesh,
        scratch_types=[
            pltpu.VMEM_SHARED((NPAD,), F32),
            pltpu.VMEM_SHARED((NPAD,), F32),
            pltpu.VMEM_SHARED((NPAD,), F32),
            pltpu.VMEM((STRIPE,), F32),
            pltpu.VMEM((STRIPE,), F32),
            pltpu.VMEM((STRIPE,), F32),
            pltpu.VMEM((STRIPE,), F32),
            pltpu.VMEM((STRIPE,), F32),
        ] + [pltpu.VMEM((C,), I32) for _ in range(6)]
          + [pltpu.VMEM((C,), F32) for _ in range(15)]
          + [pltpu.SemaphoreType.DMA for _ in range(9)],
    )

    def body(_, acc):
        return step(acc, edge1, ea1)

    acc_f = lax.fori_loop(0, itr, body, acc0)

    fin = pl.kernel(
        _fin_body,
        out_type=jax.ShapeDtypeStruct((NPAD,), F32),
        mesh=mesh,
        scratch_types=[
            pltpu.VMEM((STRIPE,), F32),
            pltpu.VMEM((STRIPE,), F32),
            pltpu.VMEM((STRIPE,), F32),
            pltpu.VMEM((STRIPE,), F32),
        ],
    )
    sig_out = fin(acc_f)
    return sig_out[:N].reshape(N, 1)
